# baseline (speedup 1.0000x reference)
import jax
import jax.numpy as jnp
from jax.experimental import pallas as pl

DIM = 64
SUPPORT = 1
NL = 10
NG = 100
EPS_BN = 1e-5
EPS_NORM = 1e-12


def _nbr_index(vertices, neighbor_num):
    inner = jnp.einsum('bvd,bwd->bvw', vertices, vertices)
    quadratic = jnp.sum(vertices ** 2, axis=2)
    distance = inner * (-2.0) + quadratic[:, None, :] + quadratic[:, :, None]
    _, idx = jax.lax.top_k(-distance, neighbor_num + 1)
    return idx[:, :, 1:]


def _index_nbr(tensor, index):
    return jax.vmap(lambda t, i: t[i])(tensor, index)


def _normalize(x, axis):
    n = jnp.linalg.norm(x, axis=axis, keepdims=True)
    return x / jnp.maximum(n, EPS_NORM)


def _conv(neighbor_index, vertices, feature_map, weights, bias, directions):
    bs, v, n = neighbor_index.shape
    neighbors = _index_nbr(vertices, neighbor_index)
    neighbor_direction = neighbors - vertices[:, :, None, :]
    nd_norm = _normalize(neighbor_direction, -1)
    sd_norm = _normalize(directions, 0)
    theta = nd_norm @ sd_norm
    feature_out = feature_map @ weights + bias
    feature_center = feature_out[:, :, :DIM]
    feature_support = feature_out[:, :, DIM:]
    feature_support = _index_nbr(feature_support, neighbor_index)
    activation_support = theta * feature_support
    activation_support = activation_support.reshape(bs, v, n, SUPPORT, DIM)
    activation_support = jnp.max(activation_support, axis=2)
    activation_support = jnp.sum(activation_support, axis=2)
    return feature_center + activation_support


def _bn(x, gamma, beta):
    mean = jnp.mean(x, axis=(0, 1), keepdims=True)
    var = jnp.var(x, axis=(0, 1), keepdims=True)
    return gamma * (x - mean) / jnp.sqrt(var + EPS_BN) + beta


def _copy_k(x_ref, o_ref):
    o_ref[...] = x_ref[...]


def kernel(vertices, input, W_l, b_l, d_l, W_g0, b_g0, d_g0, W_g1, b_g1, d_g1, gamma_l, beta_l, gamma_g0, beta_g0, gamma_g1, beta_g1):
    ni_l = _nbr_index(vertices, NL)
    ni_g = _nbr_index(vertices, NG)
    fm_l = jax.nn.relu(_bn(_conv(ni_l, vertices, input, W_l, b_l, d_l), gamma_l, beta_l))
    fm_g = jax.nn.relu(_bn(_conv(ni_g, vertices, input, W_g0, b_g0, d_g0), gamma_g0, beta_g0))
    fm_g = jax.nn.relu(_bn(_conv(ni_g, vertices, fm_g, W_g1, b_g1, d_g1), gamma_g1, beta_g1))
    out = jnp.concatenate((fm_l, fm_g), axis=2)
    return pl.pallas_call(
        _copy_k, out_shape=jax.ShapeDtypeStruct(out.shape, out.dtype))(out)


# speedup vs baseline: 1.2031x; 1.2031x over previous
import functools

import jax
import jax.numpy as jnp
from jax import lax
from jax.experimental import pallas as pl

DIM = 64
NL = 10
NG = 100
EPS_BN = 1e-5
EPS_NORM = 1e-12
V = 1024
BS = 4
IB = 8
NEG = -3.0e38


def _keys_of(d):
    u = lax.bitcast_convert_type(d, jnp.uint32)
    top = jnp.uint32(0x80000000)
    u2 = jnp.where(u >= top, ~u, u | top)
    return lax.bitcast_convert_type(u2 ^ top, jnp.int32)


def _select_mask(key, iota_j, K):
    lo = jnp.full((V, 1), jnp.int32(-(2**31)))
    hi = jnp.full((V, 1), jnp.int32(2**31 - 1))

    def body(_, c):
        lo, hi = c
        mid = (lo >> 1) + (hi >> 1) + (lo & hi & 1)
        cnt = jnp.sum((key <= mid).astype(jnp.int32), axis=1, keepdims=True)
        ge = cnt >= K
        return (jnp.where(ge, lo, mid + 1), jnp.where(ge, mid, hi))

    lo, hi = lax.fori_loop(0, 32, body, (lo, hi))
    t = lo
    m = jnp.sum((key < t).astype(jnp.int32), axis=1, keepdims=True)
    need = K - m
    eq = key == t
    lo2 = jnp.zeros((V, 1), jnp.int32)
    hi2 = jnp.full((V, 1), 1023, jnp.int32)

    def body2(_, c):
        lo2, hi2 = c
        mid = (lo2 + hi2) >> 1
        cnt = jnp.sum((eq & (iota_j <= mid)).astype(jnp.int32), axis=1,
                      keepdims=True)
        ge = cnt >= need
        return (jnp.where(ge, lo2, mid + 1), jnp.where(ge, mid, hi2))

    lo2, _ = lax.fori_loop(0, 10, body2, (lo2, hi2))
    return (key < t) | (eq & (iota_j <= lo2))


def _conv_dense(v_ref, v, x, w, b, dirs, sel_ref, out_ref):
    fo = jnp.dot(x, w, preferred_element_type=jnp.float32) + b
    fc = fo[:, :DIM]
    fs = fo[:, DIM:]
    sn = jnp.sqrt(jnp.sum(dirs * dirs, axis=0, keepdims=True))
    sd = dirs / jnp.maximum(sn, EPS_NORM)
    sd0 = sd[0:1, :][None]
    sd1 = sd[1:2, :][None]
    sd2 = sd[2:3, :][None]
    fsb = fs[None, :, :]
    vx = v[:, 0:1].reshape(1, V)
    vy = v[:, 1:2].reshape(1, V)
    vz = v[:, 2:3].reshape(1, V)

    def blk(ib, _):
        i0 = ib * IB
        vi = v_ref[0, pl.ds(i0, IB), :]
        dx = vx - vi[:, 0:1]
        dy = vy - vi[:, 1:2]
        dz = vz - vi[:, 2:3]
        rr = dx * dx + dy * dy + dz * dz
        inv = 1.0 / jnp.maximum(jnp.sqrt(rr), EPS_NORM)
        ndx = (dx * inv)[:, :, None]
        ndy = (dy * inv)[:, :, None]
        ndz = (dz * inv)[:, :, None]
        th = ndx * sd0 + ndy * sd1 + ndz * sd2
        act = th * fsb
        mf = sel_ref[0, pl.ds(i0, IB), :].astype(jnp.float32)[:, :, None]
        mx = jnp.max(jnp.where(mf > 0.0, act, NEG), axis=1)
        out_ref[0, pl.ds(i0, IB), :] = mx
        return 0

    lax.fori_loop(0, V // IB, blk, 0)
    out_ref[0] = out_ref[0] + fc


def _ka(v_ref, x_ref, wl_ref, bl_ref, dl_ref, wg_ref, bg_ref, dg_ref,
        rawl_ref, rawg_ref, selg_ref, sell_ref):
    v = v_ref[0]
    x = x_ref[0]
    q = jnp.sum(v * v, axis=1, keepdims=True)
    qt = q.reshape(1, V)
    inner = lax.dot_general(v, v, (((1,), (1,)), ((), ())),
                            preferred_element_type=jnp.float32)
    d = (inner * (-2.0) + qt) + q
    key = _keys_of(d)
    iota_j = lax.broadcasted_iota(jnp.int32, (V, V), 1)
    kmin = jnp.min(key, axis=1, keepdims=True)
    jmin = jnp.min(jnp.where(key == kmin, iota_j, V), axis=1, keepdims=True)
    notself = iota_j != jmin
    sell_ref[0] = (_select_mask(key, iota_j, NL + 1) & notself).astype(jnp.int32)
    selg_ref[0] = (_select_mask(key, iota_j, NG + 1) & notself).astype(jnp.int32)
    _conv_dense(v_ref, v, x, wl_ref[...], bl_ref[...], dl_ref[...],
                sell_ref, rawl_ref)
    _conv_dense(v_ref, v, x, wg_ref[...], bg_ref[...], dg_ref[...],
                selg_ref, rawg_ref)


def _kb(v_ref, x_ref, wg_ref, bg_ref, dg_ref, selg_ref, rawg_ref):
    v = v_ref[0]
    x = x_ref[0]
    _conv_dense(v_ref, v, x, wg_ref[...], bg_ref[...], dg_ref[...],
                selg_ref, rawg_ref)


def _kbn(x_ref, g_ref, b_ref, o_ref):
    x = x_ref[...]
    mean = jnp.mean(x, axis=0, keepdims=True)
    var = jnp.mean((x - mean) ** 2, axis=0, keepdims=True)
    y = g_ref[...] * (x - mean) / jnp.sqrt(var + EPS_BN) + b_ref[...]
    o_ref[...] = jnp.maximum(y, 0.0)


def _bn_relu(x, gamma, beta):
    flat = x.reshape(BS * V, DIM)
    out = pl.pallas_call(
        _kbn,
        out_shape=jax.ShapeDtypeStruct((BS * V, DIM), jnp.float32),
    )(flat, gamma.reshape(1, DIM), beta.reshape(1, DIM))
    return out.reshape(BS, V, DIM)


def _full(shape):
    return pl.BlockSpec(shape, lambda b: tuple(0 for _ in shape))


def _perb(shape):
    return pl.BlockSpec((1,) + shape, lambda b: (b,) + tuple(0 for _ in shape))


def kernel(vertices, input, W_l, b_l, d_l, W_g0, b_g0, d_g0, W_g1, b_g1,
           d_g1, gamma_l, beta_l, gamma_g0, beta_g0, gamma_g1, beta_g1):
    f32 = jnp.float32
    rawl, rawg0, selg, _ = pl.pallas_call(
        _ka,
        grid=(BS,),
        in_specs=[
            _perb((V, 3)), _perb((V, DIM)),
            _full((DIM, 2 * DIM)), _full((1, 2 * DIM)), _full((3, DIM)),
            _full((DIM, 2 * DIM)), _full((1, 2 * DIM)), _full((3, DIM)),
        ],
        out_specs=[_perb((V, DIM)), _perb((V, DIM)), _perb((V, V)),
                   _perb((V, V))],
        out_shape=[
            jax.ShapeDtypeStruct((BS, V, DIM), f32),
            jax.ShapeDtypeStruct((BS, V, DIM), f32),
            jax.ShapeDtypeStruct((BS, V, V), jnp.int32),
            jax.ShapeDtypeStruct((BS, V, V), jnp.int32),
        ],
    )(vertices, input, W_l, b_l.reshape(1, 2 * DIM), d_l,
      W_g0, b_g0.reshape(1, 2 * DIM), d_g0)

    fm_l = _bn_relu(rawl, gamma_l, beta_l)
    fm_g = _bn_relu(rawg0, gamma_g0, beta_g0)

    rawg1 = pl.pallas_call(
        _kb,
        grid=(BS,),
        in_specs=[
            _perb((V, 3)), _perb((V, DIM)),
            _full((DIM, 2 * DIM)), _full((1, 2 * DIM)), _full((3, DIM)),
            _perb((V, V)),
        ],
        out_specs=_perb((V, DIM)),
        out_shape=jax.ShapeDtypeStruct((BS, V, DIM), f32),
    )(vertices, fm_g, W_g1, b_g1.reshape(1, 2 * DIM), d_g1, selg)

    fm_g = _bn_relu(rawg1, gamma_g1, beta_g1)
    return jnp.concatenate((fm_l, fm_g), axis=2)


# speedup vs baseline: 1.2198x; 1.0138x over previous
import functools

import jax
import jax.numpy as jnp
from jax import lax
from jax.experimental import pallas as pl

DIM = 64
NL = 10
NG = 100
EPS_BN = 1e-5
EPS_NORM = 1e-12
V = 1024
BS = 4
IB = 8
NEG = -3.0e38


def _keys_of(d):
    u = lax.bitcast_convert_type(d, jnp.uint32)
    top = jnp.uint32(0x80000000)
    u2 = jnp.where(u >= top, ~u, u | top)
    return lax.bitcast_convert_type(u2 ^ top, jnp.int32)


def _select_mask(key, iota_j, K):
    lo = jnp.full((V, 1), jnp.int32(-(2**31)))
    hi = jnp.full((V, 1), jnp.int32(2**31 - 1))

    def body(_, c):
        lo, hi = c
        mid = (lo >> 1) + (hi >> 1) + (lo & hi & 1)
        cnt = jnp.sum((key <= mid).astype(jnp.int32), axis=1, keepdims=True)
        ge = cnt >= K
        return (jnp.where(ge, lo, mid + 1), jnp.where(ge, mid, hi))

    lo, hi = lax.fori_loop(0, 32, body, (lo, hi))
    t = lo
    m = jnp.sum((key < t).astype(jnp.int32), axis=1, keepdims=True)
    need = K - m
    eq = key == t
    lo2 = jnp.zeros((V, 1), jnp.int32)
    hi2 = jnp.full((V, 1), 1023, jnp.int32)

    def body2(_, c):
        lo2, hi2 = c
        mid = (lo2 + hi2) >> 1
        cnt = jnp.sum((eq & (iota_j <= mid)).astype(jnp.int32), axis=1,
                      keepdims=True)
        ge = cnt >= need
        return (jnp.where(ge, lo2, mid + 1), jnp.where(ge, mid, hi2))

    lo2, _ = lax.fori_loop(0, 10, body2, (lo2, hi2))
    return (key < t) | (eq & (iota_j <= lo2))


def _conv_dense(v_ref, v, x, w, b, dirs, sel_ref, out_ref):
    fo = jnp.dot(x, w, preferred_element_type=jnp.float32) + b
    fc = fo[:, :DIM]
    fs = fo[:, DIM:]
    sn = jnp.sqrt(jnp.sum(dirs * dirs, axis=0, keepdims=True))
    sd = dirs / jnp.maximum(sn, EPS_NORM)
    g0 = (sd[0:1, :] * fs)[None, :, :]
    g1 = (sd[1:2, :] * fs)[None, :, :]
    g2 = (sd[2:3, :] * fs)[None, :, :]
    vx = v[:, 0:1].reshape(1, V)
    vy = v[:, 1:2].reshape(1, V)
    vz = v[:, 2:3].reshape(1, V)

    def blk(ib, _):
        i0 = ib * IB
        vi = v_ref[0, pl.ds(i0, IB), :]
        dx = vx - vi[:, 0:1]
        dy = vy - vi[:, 1:2]
        dz = vz - vi[:, 2:3]
        rr = dx * dx + dy * dy + dz * dz
        inv = 1.0 / jnp.maximum(jnp.sqrt(rr), EPS_NORM)
        ndx = (dx * inv)[:, :, None]
        ndy = (dy * inv)[:, :, None]
        ndz = (dz * inv)[:, :, None]
        act = ndx * g0 + ndy * g1 + ndz * g2
        mf = sel_ref[0, pl.ds(i0, IB), :].astype(jnp.float32)[:, :, None]
        mx = jnp.max(jnp.where(mf > 0.0, act, NEG), axis=1)
        out_ref[0, pl.ds(i0, IB), :] = mx
        return 0

    lax.fori_loop(0, V // IB, blk, 0)
    out_ref[0] = out_ref[0] + fc


def _ka(v_ref, x_ref, wl_ref, bl_ref, dl_ref, wg_ref, bg_ref, dg_ref,
        rawl_ref, rawg_ref, selg_ref, sell_ref):
    v = v_ref[0]
    x = x_ref[0]
    q = jnp.sum(v * v, axis=1, keepdims=True)
    qt = q.reshape(1, V)
    inner = lax.dot_general(v, v, (((1,), (1,)), ((), ())),
                            preferred_element_type=jnp.float32)
    d = (inner * (-2.0) + qt) + q
    key = _keys_of(d)
    iota_j = lax.broadcasted_iota(jnp.int32, (V, V), 1)
    kmin = jnp.min(key, axis=1, keepdims=True)
    jmin = jnp.min(jnp.where(key == kmin, iota_j, V), axis=1, keepdims=True)
    notself = iota_j != jmin
    sell_ref[0] = (_select_mask(key, iota_j, NL + 1) & notself).astype(jnp.int32)
    selg_ref[0] = (_select_mask(key, iota_j, NG + 1) & notself).astype(jnp.int32)
    _conv_dense(v_ref, v, x, wl_ref[...], bl_ref[...], dl_ref[...],
                sell_ref, rawl_ref)
    _conv_dense(v_ref, v, x, wg_ref[...], bg_ref[...], dg_ref[...],
                selg_ref, rawg_ref)


def _kb(v_ref, x_ref, wg_ref, bg_ref, dg_ref, selg_ref, rawg_ref):
    v = v_ref[0]
    x = x_ref[0]
    _conv_dense(v_ref, v, x, wg_ref[...], bg_ref[...], dg_ref[...],
                selg_ref, rawg_ref)


def _kbn(x_ref, g_ref, b_ref, o_ref):
    x = x_ref[...]
    mean = jnp.mean(x, axis=0, keepdims=True)
    var = jnp.mean((x - mean) ** 2, axis=0, keepdims=True)
    y = g_ref[...] * (x - mean) / jnp.sqrt(var + EPS_BN) + b_ref[...]
    o_ref[...] = jnp.maximum(y, 0.0)


def _bn_relu(x, gamma, beta):
    flat = x.reshape(BS * V, DIM)
    out = pl.pallas_call(
        _kbn,
        out_shape=jax.ShapeDtypeStruct((BS * V, DIM), jnp.float32),
    )(flat, gamma.reshape(1, DIM), beta.reshape(1, DIM))
    return out.reshape(BS, V, DIM)


def _full(shape):
    return pl.BlockSpec(shape, lambda b: tuple(0 for _ in shape))


def _perb(shape):
    return pl.BlockSpec((1,) + shape, lambda b: (b,) + tuple(0 for _ in shape))


def kernel(vertices, input, W_l, b_l, d_l, W_g0, b_g0, d_g0, W_g1, b_g1,
           d_g1, gamma_l, beta_l, gamma_g0, beta_g0, gamma_g1, beta_g1):
    f32 = jnp.float32
    rawl, rawg0, selg, _ = pl.pallas_call(
        _ka,
        grid=(BS,),
        in_specs=[
            _perb((V, 3)), _perb((V, DIM)),
            _full((DIM, 2 * DIM)), _full((1, 2 * DIM)), _full((3, DIM)),
            _full((DIM, 2 * DIM)), _full((1, 2 * DIM)), _full((3, DIM)),
        ],
        out_specs=[_perb((V, DIM)), _perb((V, DIM)), _perb((V, V)),
                   _perb((V, V))],
        out_shape=[
            jax.ShapeDtypeStruct((BS, V, DIM), f32),
            jax.ShapeDtypeStruct((BS, V, DIM), f32),
            jax.ShapeDtypeStruct((BS, V, V), jnp.int32),
            jax.ShapeDtypeStruct((BS, V, V), jnp.int32),
        ],
    )(vertices, input, W_l, b_l.reshape(1, 2 * DIM), d_l,
      W_g0, b_g0.reshape(1, 2 * DIM), d_g0)

    fm_l = _bn_relu(rawl, gamma_l, beta_l)
    fm_g = _bn_relu(rawg0, gamma_g0, beta_g0)

    rawg1 = pl.pallas_call(
        _kb,
        grid=(BS,),
        in_specs=[
            _perb((V, 3)), _perb((V, DIM)),
            _full((DIM, 2 * DIM)), _full((1, 2 * DIM)), _full((3, DIM)),
            _perb((V, V)),
        ],
        out_specs=_perb((V, DIM)),
        out_shape=jax.ShapeDtypeStruct((BS, V, DIM), f32),
    )(vertices, fm_g, W_g1, b_g1.reshape(1, 2 * DIM), d_g1, selg)

    fm_g = _bn_relu(rawg1, gamma_g1, beta_g1)
    return jnp.concatenate((fm_l, fm_g), axis=2)
